# baseline (speedup 1.0000x reference)
import dataclasses
import functools

import jax
import jax.numpy as jnp
from jax import lax
from jax.experimental import pallas as pl
from jax.experimental.pallas import tpu as pltpu
from jax.experimental.pallas import tpu_sc as plsc

N = 10000
E = 320000
D = 128

NC = 2
NS = 16
NW = NC * NS
EDGES_PER_TILE = E // NW
CHUNK = 40
NCHUNK = EDGES_PER_TILE // CHUNK
NB = 3
ND = 6
TAIL = NCHUNK % ND
ROWS_A = 632
ROWS_B = N - 15 * ROWS_A


def _edge_phase(xp, src, dst, edge_attr):
    mesh = plsc.VectorSubcoreMesh(core_axis_name="c", subcore_axis_name="s")
    cp = pltpu.CompilerParams()
    if "needs_layout_passes" in pltpu.CompilerParams.__dataclass_fields__:
        cp = dataclasses.replace(cp, needs_layout_passes=False)
    if "use_tc_tiling_on_sc" in pltpu.CompilerParams.__dataclass_fields__:
        cp = dataclasses.replace(cp, use_tc_tiling_on_sc=False)

    @functools.partial(
        pl.kernel,
        out_type=jax.ShapeDtypeStruct((NC * N, D), jnp.float32),
        mesh=mesh,
        compiler_params=cp,
        scratch_types=(
            [pltpu.VMEM((CHUNK, D), jnp.bfloat16)] * NB
            + [pltpu.VMEM((CHUNK, D), jnp.float32)] * (2 * NB)
            + [pltpu.VMEM((CHUNK,), jnp.int32)] * (NB + ND)
            + [pltpu.VMEM_SHARED((N, D), jnp.float32)]
            + [pltpu.SemaphoreType.DMA] * (3 * NB + 1)
        ),
    )
    def edge_kernel(x_hbm, src_hbm, dst_hbm, ea_hbm, out_hbm, *bufs):
        rows = bufs[0:NB]
        eab = bufs[NB:2 * NB]
        msg = bufs[2 * NB:3 * NB]
        srcb = bufs[3 * NB:4 * NB]
        dstb = bufs[4 * NB:4 * NB + ND]
        aggr_sh = bufs[4 * NB + ND]
        sems = bufs[4 * NB + ND + 1:]
        si = sems[0:NB]
        sg = sems[NB:2 * NB]
        so = sems[2 * NB:3 * NB]
        sz = sems[3 * NB]

        c = lax.axis_index("c")
        s = lax.axis_index("s")
        wid = c * NS + s
        base = wid * EDGES_PER_TILE

        @pl.loop(0, CHUNK)
        def _(i):
            for j in range(0, D, 16):
                msg[0][i, pl.ds(j, 16)] = jnp.zeros((16,), jnp.float32)

        def init_copies(fn):
            @pl.when(s < 15)
            def _():
                for r in range(0, ROWS_A - ROWS_A % CHUNK, CHUNK):
                    fn(msg[0], aggr_sh.at[pl.ds(s * ROWS_A + r, CHUNK)])
                fn(msg[0].at[pl.ds(0, ROWS_A % CHUNK)],
                   aggr_sh.at[pl.ds(s * ROWS_A + ROWS_A - ROWS_A % CHUNK,
                                    ROWS_A % CHUNK)])

            @pl.when(s == 15)
            def _():
                for r in range(0, ROWS_B, CHUNK):
                    fn(msg[0], aggr_sh.at[pl.ds(15 * ROWS_A + r, CHUNK)])

        init_copies(lambda a, b: pltpu.async_copy(a, b, sz))

        def issue_idx(t, kb, kd):
            off = base + t * CHUNK
            pltpu.async_copy(src_hbm.at[pl.ds(off, CHUNK)], srcb[kb], si[kb])
            pltpu.async_copy(dst_hbm.at[pl.ds(off, CHUNK)], dstb[kd], si[kb])

        def wait_idx(t, kb, kd):
            off = base + t * CHUNK
            pltpu.make_async_copy(src_hbm.at[pl.ds(off, CHUNK)], srcb[kb],
                                  si[kb]).wait()
            pltpu.make_async_copy(dst_hbm.at[pl.ds(off, CHUNK)], dstb[kd],
                                  si[kb]).wait()

        def issue_gather(t, kb):
            pltpu.async_copy(x_hbm.at[srcb[kb]], rows[kb], sg[kb])
            pltpu.async_copy(ea_hbm.at[pl.ds(base + t * CHUNK, CHUNK)],
                             eab[kb], sg[kb])

        def wait_gather(t, kb):
            pltpu.make_async_copy(x_hbm.at[srcb[kb]], rows[kb],
                                  sg[kb]).wait()
            pltpu.make_async_copy(ea_hbm.at[pl.ds(base + t * CHUNK, CHUNK)],
                                  eab[kb], sg[kb]).wait()

        def issue_scat(kb, kd):
            pltpu.async_copy(msg[kb], aggr_sh.at[dstb[kd]], so[kb], add=True)

        def wait_scat(kb, kd):
            pltpu.make_async_copy(msg[kb], aggr_sh.at[dstb[kd]],
                                  so[kb]).wait()

        def compute(kb):
            mask_hi = jnp.full((16,), -65536, jnp.int32)

            @pl.loop(0, CHUNK)
            def _(i):
                    for k in range(0, D // 32):
                        w = plsc.bitcast(
                            rows[kb][i, pl.ds(32 * k, 32)], jnp.int32)
                        lo = plsc.bitcast(w << 16, jnp.float32)
                        hi = plsc.bitcast(w & mask_hi, jnp.float32)
                        sl = (i, pl.ds(32 * k, 16))
                        sh = (i, pl.ds(32 * k + 16, 16))
                        msg[kb][sl] = jnp.maximum(lo + eab[kb][sl], 0.0)
                        msg[kb][sh] = jnp.maximum(hi + eab[kb][sh], 0.0)

        def slot(t, k, first):
            kb = k % NB
            wait_gather(t, kb)
            if first:
                @pl.when(t >= NB)
                def _():
                    wait_scat(kb, (k + NB) % ND)
            else:
                wait_scat(kb, (k + NB) % ND)
            compute(kb)
            issue_scat(kb, k)
            if not isinstance(t, int) or t + NB < NCHUNK:
                issue_idx(t + NB, kb, (k + NB) % ND)
            if not isinstance(t, int) or t + 2 < NCHUNK:
                kb2 = (k + 2) % NB
                wait_idx(t + 2, kb2, (k + 2) % ND)
                issue_gather(t + 2, kb2)

        for t0 in range(NB):
            issue_idx(t0, t0 % NB, t0 % ND)
        for t0 in range(2):
            wait_idx(t0, t0 % NB, t0 % ND)
            issue_gather(t0, t0 % NB)

        init_copies(lambda a, b: pltpu.make_async_copy(a, b, sz).wait())
        plsc.subcore_barrier()

        @pl.loop(0, (NCHUNK - TAIL) // ND)
        def _(g):
            for k in range(ND):
                slot(g * ND + k, k, first=(k < NB))

        for t in range(NCHUNK - TAIL, NCHUNK):
            slot(t, t % ND, first=False)

        for t in range(NCHUNK - NB, NCHUNK):
            wait_scat(t % NB, t % ND)

        plsc.subcore_barrier()

        @pl.when(s < 15)
        def _():
            pltpu.sync_copy(
                aggr_sh.at[pl.ds(s * ROWS_A, ROWS_A)],
                out_hbm.at[pl.ds(c * N + s * ROWS_A, ROWS_A)])

        @pl.when(s == 15)
        def _():
            pltpu.sync_copy(
                aggr_sh.at[pl.ds(15 * ROWS_A, ROWS_B)],
                out_hbm.at[pl.ds(c * N + 15 * ROWS_A, ROWS_B)])

    return edge_kernel(xp, src, dst, edge_attr)


def _dense_body(xb, pb, gw, gb, w1, b1, w2, b2,
                s0, t0, s1, t1, s2, t2, ob):
    z = xb[...] + pb[0] + pb[1]
    h = jnp.maximum(z * s0[...] + t0[...], 0.0)
    hg = jnp.dot(h, gw[...], preferred_element_type=jnp.float32) + gb[...]
    h2 = (xb[...] + hg) * s1[...] + t1[...]
    hf = jnp.maximum(
        jnp.dot(h2, w1[...], preferred_element_type=jnp.float32) + b1[...],
        0.0)
    hf = jnp.dot(hf, w2[...], preferred_element_type=jnp.float32) + b2[...]
    ob[...] = (h2 + hf) * s2[...] + t2[...]


_BN = 1000


def _dense_phase(x, parts, gin_W, gin_b, ffn_W1, ffn_b1, ffn_W2, ffn_b2,
                 s0, t0, s1, t1, s2, t2):
    full = lambda shape: pl.BlockSpec(shape, lambda i: (0,) * len(shape))
    return pl.pallas_call(
        _dense_body,
        grid=(N // _BN,),
        in_specs=[
            pl.BlockSpec((_BN, D), lambda i: (i, 0)),
            pl.BlockSpec((NC, _BN, D), lambda i: (0, i, 0)),
            full((D, D)), full((1, D)),
            full((D, 2 * D)), full((1, 2 * D)),
            full((2 * D, D)), full((1, D)),
            full((1, D)), full((1, D)), full((1, D)),
            full((1, D)), full((1, D)), full((1, D)),
        ],
        out_specs=pl.BlockSpec((_BN, D), lambda i: (i, 0)),
        out_shape=jax.ShapeDtypeStruct((N, D), jnp.float32),
    )(x, parts, gin_W, gin_b, ffn_W1, ffn_b1, ffn_W2, ffn_b2,
      s0, t0, s1, t1, s2, t2)


def kernel(x, edge_index, edge_attr,
           gin_bn_g, gin_bn_b, gin_bn_rm, gin_bn_rv, gin_W, gin_b,
           n1_g, n1_b, n1_rm, n1_rv,
           ffn_W1, ffn_b1, ffn_W2, ffn_b2,
           n2_g, n2_b, n2_rm, n2_rv):
    src = edge_index[0]
    dst = edge_index[1]
    xp = (x.reshape(N, D // 32, 2, 16).transpose(0, 1, 3, 2)
          .reshape(N, D).astype(jnp.bfloat16))

    parts = _edge_phase(xp, src, dst, edge_attr).reshape(NC, N, D)

    def fold(g, b, rm, rv):
        s = (g * lax.rsqrt(rv + 1e-5)).reshape(1, -1)
        return s, (b - rm * s[0]).reshape(1, -1)

    s0, t0 = fold(gin_bn_g, gin_bn_b, gin_bn_rm, gin_bn_rv)
    s1, t1 = fold(n1_g, n1_b, n1_rm, n1_rv)
    s2, t2 = fold(n2_g, n2_b, n2_rm, n2_rv)

    return _dense_phase(x, parts, gin_W, gin_b.reshape(1, -1),
                        ffn_W1, ffn_b1.reshape(1, -1),
                        ffn_W2, ffn_b2.reshape(1, -1),
                        s0, t0, s1, t1, s2, t2)


# speedup vs baseline: 1.3436x; 1.3436x over previous
import dataclasses
import functools

import jax
import jax.numpy as jnp
from jax import lax
from jax.experimental import pallas as pl
from jax.experimental.pallas import tpu as pltpu
from jax.experimental.pallas import tpu_sc as plsc

N = 10000
E = 320000
D = 128

NC = 2
NS = 16
NW = NC * NS
EDGES_PER_TILE = E // NW
CHUNK = 40
NCHUNK = EDGES_PER_TILE // CHUNK
NB = 3
ND = 6
TAIL = NCHUNK % ND
ROWS_A = 632
ROWS_B = N - 15 * ROWS_A


def _edge_phase(x, src, dst, edge_attr):
    mesh = plsc.VectorSubcoreMesh(core_axis_name="c", subcore_axis_name="s")
    cp = pltpu.CompilerParams()

    @functools.partial(
        pl.kernel,
        out_type=jax.ShapeDtypeStruct((NC * N, D), jnp.float32),
        mesh=mesh,
        compiler_params=cp,
        scratch_types=(
            [pltpu.VMEM((CHUNK, D), jnp.float32)] * NB
            + [pltpu.VMEM((CHUNK, D), jnp.float32)] * (2 * NB)
            + [pltpu.VMEM((CHUNK,), jnp.int32)] * (NB + ND)
            + [pltpu.VMEM_SHARED((N, D), jnp.float32)]
            + [pltpu.SemaphoreType.DMA] * (3 * NB + 1)
        ),
    )
    def edge_kernel(x_hbm, src_hbm, dst_hbm, ea_hbm, out_hbm, *bufs):
        rows = bufs[0:NB]
        eab = bufs[NB:2 * NB]
        msg = bufs[2 * NB:3 * NB]
        srcb = bufs[3 * NB:4 * NB]
        dstb = bufs[4 * NB:4 * NB + ND]
        aggr_sh = bufs[4 * NB + ND]
        sems = bufs[4 * NB + ND + 1:]
        si = sems[0:NB]
        sg = sems[NB:2 * NB]
        so = sems[2 * NB:3 * NB]
        sz = sems[3 * NB]

        c = lax.axis_index("c")
        s = lax.axis_index("s")
        wid = c * NS + s
        base = wid * EDGES_PER_TILE

        @pl.loop(0, CHUNK)
        def _(i):
            for j in range(0, D, 16):
                msg[0][i, pl.ds(j, 16)] = jnp.zeros((16,), jnp.float32)

        def init_copies(fn):
            @pl.when(s < 15)
            def _():
                for r in range(0, ROWS_A - ROWS_A % CHUNK, CHUNK):
                    fn(msg[0], aggr_sh.at[pl.ds(s * ROWS_A + r, CHUNK)])
                fn(msg[0].at[pl.ds(0, ROWS_A % CHUNK)],
                   aggr_sh.at[pl.ds(s * ROWS_A + ROWS_A - ROWS_A % CHUNK,
                                    ROWS_A % CHUNK)])

            @pl.when(s == 15)
            def _():
                for r in range(0, ROWS_B, CHUNK):
                    fn(msg[0], aggr_sh.at[pl.ds(15 * ROWS_A + r, CHUNK)])

        init_copies(lambda a, b: pltpu.async_copy(a, b, sz))

        def issue_idx(t, kb, kd):
            off = base + t * CHUNK
            pltpu.async_copy(src_hbm.at[pl.ds(off, CHUNK)], srcb[kb], si[kb])
            pltpu.async_copy(dst_hbm.at[pl.ds(off, CHUNK)], dstb[kd], si[kb])

        def wait_idx(t, kb, kd):
            off = base + t * CHUNK
            pltpu.make_async_copy(src_hbm.at[pl.ds(off, CHUNK)], srcb[kb],
                                  si[kb]).wait()
            pltpu.make_async_copy(dst_hbm.at[pl.ds(off, CHUNK)], dstb[kd],
                                  si[kb]).wait()

        def issue_gather(t, kb):
            pltpu.async_copy(x_hbm.at[srcb[kb]], rows[kb], sg[kb])
            pltpu.async_copy(ea_hbm.at[pl.ds(base + t * CHUNK, CHUNK)],
                             eab[kb], sg[kb])

        def wait_gather(t, kb):
            pltpu.make_async_copy(x_hbm.at[srcb[kb]], rows[kb],
                                  sg[kb]).wait()
            pltpu.make_async_copy(ea_hbm.at[pl.ds(base + t * CHUNK, CHUNK)],
                                  eab[kb], sg[kb]).wait()

        def issue_scat(kb, kd):
            pltpu.async_copy(msg[kb], aggr_sh.at[dstb[kd]], so[kb], add=True)

        def wait_scat(kb, kd):
            pltpu.make_async_copy(msg[kb], aggr_sh.at[dstb[kd]],
                                  so[kb]).wait()

        def compute(kb):
            @pl.loop(0, CHUNK)
            def _(i):
                for j in range(0, D, 16):
                    sl = (i, pl.ds(j, 16))
                    msg[kb][sl] = jnp.maximum(
                        rows[kb][sl] + eab[kb][sl], 0.0)

        def slot(t, k, first):
            kb = k % NB
            wait_gather(t, kb)
            if first:
                @pl.when(t >= NB)
                def _():
                    wait_scat(kb, (k + NB) % ND)
            else:
                wait_scat(kb, (k + NB) % ND)
            compute(kb)
            issue_scat(kb, k)
            if not isinstance(t, int) or t + NB < NCHUNK:
                issue_idx(t + NB, kb, (k + NB) % ND)
            if not isinstance(t, int) or t + 2 < NCHUNK:
                kb2 = (k + 2) % NB
                wait_idx(t + 2, kb2, (k + 2) % ND)
                issue_gather(t + 2, kb2)

        for t0 in range(NB):
            issue_idx(t0, t0 % NB, t0 % ND)
        for t0 in range(2):
            wait_idx(t0, t0 % NB, t0 % ND)
            issue_gather(t0, t0 % NB)

        init_copies(lambda a, b: pltpu.make_async_copy(a, b, sz).wait())
        plsc.subcore_barrier()

        @pl.loop(0, (NCHUNK - TAIL) // ND)
        def _(g):
            for k in range(ND):
                slot(g * ND + k, k, first=(k < NB))

        for t in range(NCHUNK - TAIL, NCHUNK):
            slot(t, t % ND, first=False)

        for t in range(NCHUNK - NB, NCHUNK):
            wait_scat(t % NB, t % ND)

        plsc.subcore_barrier()

        @pl.when(s < 15)
        def _():
            pltpu.sync_copy(
                aggr_sh.at[pl.ds(s * ROWS_A, ROWS_A)],
                out_hbm.at[pl.ds(c * N + s * ROWS_A, ROWS_A)])

        @pl.when(s == 15)
        def _():
            pltpu.sync_copy(
                aggr_sh.at[pl.ds(15 * ROWS_A, ROWS_B)],
                out_hbm.at[pl.ds(c * N + 15 * ROWS_A, ROWS_B)])

    return edge_kernel(x, src, dst, edge_attr)


def _dense_body(xb, pb, gw, gb, w1, b1, w2, b2,
                s0, t0, s1, t1, s2, t2, ob):
    z = xb[...] + pb[0] + pb[1]
    h = jnp.maximum(z * s0[...] + t0[...], 0.0)
    hg = jnp.dot(h, gw[...], preferred_element_type=jnp.float32) + gb[...]
    h2 = (xb[...] + hg) * s1[...] + t1[...]
    hf = jnp.maximum(
        jnp.dot(h2, w1[...], preferred_element_type=jnp.float32) + b1[...],
        0.0)
    hf = jnp.dot(hf, w2[...], preferred_element_type=jnp.float32) + b2[...]
    ob[...] = (h2 + hf) * s2[...] + t2[...]


_BN = 1000


def _dense_phase(x, parts, gin_W, gin_b, ffn_W1, ffn_b1, ffn_W2, ffn_b2,
                 s0, t0, s1, t1, s2, t2):
    full = lambda shape: pl.BlockSpec(shape, lambda i: (0,) * len(shape))
    return pl.pallas_call(
        _dense_body,
        grid=(N // _BN,),
        in_specs=[
            pl.BlockSpec((_BN, D), lambda i: (i, 0)),
            pl.BlockSpec((NC, _BN, D), lambda i: (0, i, 0)),
            full((D, D)), full((1, D)),
            full((D, 2 * D)), full((1, 2 * D)),
            full((2 * D, D)), full((1, D)),
            full((1, D)), full((1, D)), full((1, D)),
            full((1, D)), full((1, D)), full((1, D)),
        ],
        out_specs=pl.BlockSpec((_BN, D), lambda i: (i, 0)),
        out_shape=jax.ShapeDtypeStruct((N, D), jnp.float32),
    )(x, parts, gin_W, gin_b, ffn_W1, ffn_b1, ffn_W2, ffn_b2,
      s0, t0, s1, t1, s2, t2)


def kernel(x, edge_index, edge_attr,
           gin_bn_g, gin_bn_b, gin_bn_rm, gin_bn_rv, gin_W, gin_b,
           n1_g, n1_b, n1_rm, n1_rv,
           ffn_W1, ffn_b1, ffn_W2, ffn_b2,
           n2_g, n2_b, n2_rm, n2_rv):
    src = edge_index[0]
    dst = edge_index[1]
    parts = _edge_phase(x, src, dst, edge_attr).reshape(NC, N, D)

    def fold(g, b, rm, rv):
        s = (g * lax.rsqrt(rv + 1e-5)).reshape(1, -1)
        return s, (b - rm * s[0]).reshape(1, -1)

    s0, t0 = fold(gin_bn_g, gin_bn_b, gin_bn_rm, gin_bn_rv)
    s1, t1 = fold(n1_g, n1_b, n1_rm, n1_rv)
    s2, t2 = fold(n2_g, n2_b, n2_rm, n2_rv)

    return _dense_phase(x, parts, gin_W, gin_b.reshape(1, -1),
                        ffn_W1, ffn_b1.reshape(1, -1),
                        ffn_W2, ffn_b2.reshape(1, -1),
                        s0, t0, s1, t1, s2, t2)
